# baseline (speedup 1.0000x reference)
import math

import jax
import jax.numpy as jnp
from jax import lax
from jax.experimental import pallas as pl
from jax.experimental.pallas import tpu as pltpu

_B = 65536
_W = 128
_H = _B // _W

_ATAN_DEG_COEF = (
    57.295746061603964,
    -19.096811113331427,
    11.431149564673765,
    -7.983180088342099,
    5.563125122822412,
    -3.2591483873916567,
    1.2930681400742676,
    -0.24395644403106906,
)


def _zone_from_xy(x, y):
    ax = jnp.abs(x)
    ay = jnp.abs(y)
    den = jnp.maximum(ax, ay)
    num = jnp.minimum(ax, ay)
    safe_den = jnp.maximum(den, jnp.float32(1e-30))
    r = num / safe_den
    r2 = r * r
    acc = jnp.float32(_ATAN_DEG_COEF[-1])
    for c in _ATAN_DEG_COEF[-2::-1]:
        acc = acc * r2 + jnp.float32(c)
    pd = acc * r
    sx = x < 0.0
    sy = y < 0.0
    sw = ay > ax
    sneg = jnp.logical_xor(jnp.logical_xor(sx, sy), sw)
    spd = jnp.where(sneg, -pd, pd)
    cpos = jnp.where(sx, jnp.where(sw, 191.0, 281.0),
                     jnp.where(sw, 191.0, 101.0))
    cneg = jnp.where(sx, jnp.where(sw, 371.0, 281.0),
                     jnp.where(sw, 11.0, 101.0))
    c0 = jnp.where(sy, cneg, cpos).astype(jnp.float32)
    return ((c0 + spd) * jnp.float32(1.0 / 22.0)).astype(jnp.int32)


def _body(z_ref, d_ref, out_ref):
    zt = jnp.transpose(z_ref[0], (1, 0, 2))
    dt = jnp.transpose(d_ref[0], (1, 0, 2))
    x = zt[3] - zt[1]
    y = zt[2] - zt[4]
    zone = _zone_from_xy(x, y)
    b0 = jnp.bitwise_and(zone, 1) != 0
    b1 = jnp.bitwise_and(zone, 2) != 0
    b2 = jnp.bitwise_and(zone, 4) != 0
    v01 = jnp.where(b0, dt[1], dt[0])
    v23 = jnp.where(b0, dt[3], dt[2])
    v45 = jnp.where(b0, dt[5], dt[4])
    v67 = jnp.where(b0, dt[7], dt[6])
    v03 = jnp.where(b1, v23, v01)
    v47 = jnp.where(b1, v67, v45)
    val = jnp.where(b2, v47, v03)
    out_ref[...] = jnp.where(zt[0] == 0.0, jnp.float32(0.0), val)


@jax.jit
def kernel(z_1, dir):
    z4 = z_1.reshape(_H, _W, 2, 8).transpose(2, 0, 3, 1)
    d4 = dir.reshape(_H, _W, 1, 8).transpose(2, 0, 3, 1)
    rows = 128
    grid = _H // rows
    spec = pl.BlockSpec((1, rows, 8, _W), lambda i: (0, i, 0, 0))
    out = pl.pallas_call(
        _body,
        grid=(grid,),
        in_specs=[spec, spec],
        out_specs=pl.BlockSpec((rows, _W), lambda i: (i, 0)),
        out_shape=jax.ShapeDtypeStruct((_H, _W), jnp.float32),
        compiler_params=pltpu.CompilerParams(
            dimension_semantics=("arbitrary",)),
    )(z4, d4)
    return out.reshape(_B)


# speedup vs baseline: 1.2810x; 1.2810x over previous
import math

import jax
import jax.numpy as jnp
from jax import lax
from jax.experimental import pallas as pl
from jax.experimental.pallas import tpu as pltpu

_B = 65536
_W = 128
_H = _B // _W

_ATAN_DEG_COEF = (
    57.295746061603964,
    -19.096811113331427,
    11.431149564673765,
    -7.983180088342099,
    5.563125122822412,
    -3.2591483873916567,
    1.2930681400742676,
    -0.24395644403106906,
)


def _zone_from_xy(x, y):
    ax = jnp.abs(x)
    ay = jnp.abs(y)
    den = jnp.maximum(ax, ay)
    num = jnp.minimum(ax, ay)
    safe_den = jnp.maximum(den, jnp.float32(1e-30))
    r = num / safe_den
    r2 = r * r
    acc = jnp.float32(_ATAN_DEG_COEF[-1])
    for c in _ATAN_DEG_COEF[-2::-1]:
        acc = acc * r2 + jnp.float32(c)
    pd = acc * r
    sx = x < 0.0
    sy = y < 0.0
    sw = ay > ax
    sneg = jnp.logical_xor(jnp.logical_xor(sx, sy), sw)
    spd = jnp.where(sneg, -pd, pd)
    cpos = jnp.where(sx, jnp.where(sw, 191.0, 281.0),
                     jnp.where(sw, 191.0, 101.0))
    cneg = jnp.where(sx, jnp.where(sw, 371.0, 281.0),
                     jnp.where(sw, 11.0, 101.0))
    c0 = jnp.where(sy, cneg, cpos).astype(jnp.float32)
    return ((c0 + spd) * jnp.float32(1.0 / 22.0)).astype(jnp.int32)


def _body(z_ref, d_ref, out_ref):
    zt = jnp.transpose(z_ref[0], (1, 0, 2))
    dt = jnp.transpose(d_ref[0], (1, 0, 2))
    x = zt[3] - zt[1]
    y = zt[2] - zt[4]
    zone = _zone_from_xy(x, y)
    b0 = jnp.bitwise_and(zone, 1) != 0
    b1 = jnp.bitwise_and(zone, 2) != 0
    b2 = jnp.bitwise_and(zone, 4) != 0
    v01 = jnp.where(b0, dt[1], dt[0])
    v23 = jnp.where(b0, dt[3], dt[2])
    v45 = jnp.where(b0, dt[5], dt[4])
    v67 = jnp.where(b0, dt[7], dt[6])
    v03 = jnp.where(b1, v23, v01)
    v47 = jnp.where(b1, v67, v45)
    val = jnp.where(b2, v47, v03)
    out_ref[...] = jnp.where(zt[0] == 0.0, jnp.float32(0.0), val)


@jax.jit
def kernel(z_1, dir):
    z4 = z_1.reshape(_H, _W, 2, 8).transpose(2, 0, 3, 1)
    d4 = dir.reshape(_H, _W, 1, 8).transpose(2, 0, 3, 1)
    rows = 256
    grid = _H // rows
    spec = pl.BlockSpec((1, rows, 8, _W), lambda i: (0, i, 0, 0))
    out = pl.pallas_call(
        _body,
        grid=(grid,),
        in_specs=[spec, spec],
        out_specs=pl.BlockSpec((rows, _W), lambda i: (i, 0)),
        out_shape=jax.ShapeDtypeStruct((_H, _W), jnp.float32),
        compiler_params=pltpu.CompilerParams(
            dimension_semantics=("arbitrary",)),
    )(z4, d4)
    return out.reshape(_B)
